# baseline (speedup 1.0000x reference)
import jax
import jax.numpy as jnp
from jax.experimental import pallas as pl
from jax.experimental.pallas import tpu as pltpu

_MARGIN = 1.0
_BR = 256
_BIG = 32768.0
_NLABELS = 128


def _triplet_tile(u_ref, v_ref, vself_ref, s0_ref, out_ref):
    i = pl.program_id(0)
    u = u_ref[...]
    v = v_ref[...]
    vself = vself_ref[...]
    s0 = s0_ref[...]

    m = jax.lax.dot_general(
        u, v, (((1,), (1,)), ((), ())), preferred_element_type=jnp.float32
    )

    mp = jnp.max(m, axis=1, keepdims=True)
    mn = jnp.min(m, axis=1, keepdims=True)

    d2_0 = m[:, 0:1] - s0

    ap2 = jnp.where(mp < vself + 0.5, d2_0, mp - _BIG)
    an2 = jnp.where(mn > 0.5 * _BIG, d2_0, mn)

    ap = jnp.sqrt(jnp.maximum(ap2, 0.0) + 1e-12)
    an = jnp.sqrt(jnp.maximum(an2, 0.0) + 1e-12)
    loss = jnp.maximum(ap - an + _MARGIN, 0.0)
    psum = jnp.sum(loss, axis=(0, 1), keepdims=True)

    @pl.when(i == 0)
    def _():
        out_ref[...] = jnp.zeros((1, 1), jnp.float32)

    out_ref[...] += psum


def kernel(x, target):
    n, d = x.shape
    target = target.astype(jnp.int32)
    x_bf = x.astype(jnp.bfloat16)
    sq = jnp.sum(x * x, axis=1)
    sq_hi = sq.astype(jnp.bfloat16)
    sq_lo = (sq - sq_hi.astype(jnp.float32)).astype(jnp.bfloat16)
    one = jnp.ones((n, 1), jnp.bfloat16)
    onehot = jax.nn.one_hot(target, _NLABELS, dtype=jnp.bfloat16)
    u = jnp.concatenate(
        [-2.0 * x_bf, sq_hi[:, None], sq_lo[:, None], one, one,
         _BIG * onehot], axis=1)
    v = jnp.concatenate(
        [x_bf, one, one, sq_hi[:, None], sq_lo[:, None], onehot], axis=1)
    k = d + 4 + _NLABELS
    vself = jnp.sum(u.astype(jnp.float32) * v.astype(jnp.float32),
                    axis=1, keepdims=True)
    s0 = (_BIG * (target == target[0]).astype(jnp.float32)).reshape(n, 1)
    grid = (n // _BR,)

    total = pl.pallas_call(
        _triplet_tile,
        grid=grid,
        in_specs=[
            pl.BlockSpec((_BR, k), lambda i: (i, 0)),
            pl.BlockSpec((n, k), lambda i: (0, 0)),
            pl.BlockSpec((_BR, 1), lambda i: (i, 0)),
            pl.BlockSpec((_BR, 1), lambda i: (i, 0)),
        ],
        out_specs=pl.BlockSpec((1, 1), lambda i: (0, 0)),
        out_shape=jax.ShapeDtypeStruct((1, 1), jnp.float32),
        compiler_params=pltpu.CompilerParams(
            dimension_semantics=("arbitrary",),
        ),
    )(u, v, vself, s0)

    loss_mean = total[0, 0] / n
    return (loss_mean, jnp.asarray(n, dtype=jnp.int32))


# speedup vs baseline: 1.0574x; 1.0574x over previous
import jax
import jax.numpy as jnp
from jax.experimental import pallas as pl
from jax.experimental.pallas import tpu as pltpu

_MARGIN = 1.0
_BH = 256
_BR = 512
_BIG = 32768.0
_NLABELS = 128


def _mine_half(u, v, vself, s0):
    m = jax.lax.dot_general(
        u, v, (((1,), (1,)), ((), ())), preferred_element_type=jnp.float32
    )
    mp = jnp.max(m, axis=1, keepdims=True)
    mn = jnp.min(m, axis=1, keepdims=True)
    d2_0 = m[:, 0:1] - s0
    ap2 = jnp.where(mp < vself + 0.5, d2_0, mp - _BIG)
    an2 = jnp.where(mn > 0.5 * _BIG, d2_0, mn)
    ap = jnp.sqrt(jnp.maximum(ap2, 0.0) + 1e-12)
    an = jnp.sqrt(jnp.maximum(an2, 0.0) + 1e-12)
    loss = jnp.maximum(ap - an + _MARGIN, 0.0)
    return jnp.sum(loss, axis=(0, 1), keepdims=True)


def _triplet_tile(u_ref, v_ref, vself_ref, s0_ref, out_ref):
    i = pl.program_id(0)
    v = v_ref[...]

    psum0 = _mine_half(u_ref[0:_BH, :], v,
                       vself_ref[0:_BH, :], s0_ref[0:_BH, :])
    psum1 = _mine_half(u_ref[_BH:_BR, :], v,
                       vself_ref[_BH:_BR, :], s0_ref[_BH:_BR, :])

    @pl.when(i == 0)
    def _():
        out_ref[...] = jnp.zeros((1, 1), jnp.float32)

    out_ref[...] += psum0 + psum1


def kernel(x, target):
    n, d = x.shape
    target = target.astype(jnp.int32)
    x_bf = x.astype(jnp.bfloat16)
    sq = jnp.sum(x * x, axis=1)
    sq_hi = sq.astype(jnp.bfloat16)
    sq_lo = (sq - sq_hi.astype(jnp.float32)).astype(jnp.bfloat16)
    one = jnp.ones((n, 1), jnp.bfloat16)
    onehot = jax.nn.one_hot(target, _NLABELS, dtype=jnp.bfloat16)
    u = jnp.concatenate(
        [-2.0 * x_bf, sq_hi[:, None], sq_lo[:, None], one, one,
         _BIG * onehot], axis=1)
    v = jnp.concatenate(
        [x_bf, one, one, sq_hi[:, None], sq_lo[:, None], onehot], axis=1)
    k = d + 4 + _NLABELS
    vself = jnp.sum(u.astype(jnp.float32) * v.astype(jnp.float32),
                    axis=1, keepdims=True)
    s0 = (_BIG * (target == target[0]).astype(jnp.float32)).reshape(n, 1)
    grid = (n // _BR,)

    total = pl.pallas_call(
        _triplet_tile,
        grid=grid,
        in_specs=[
            pl.BlockSpec((_BR, k), lambda i: (i, 0)),
            pl.BlockSpec((n, k), lambda i: (0, 0)),
            pl.BlockSpec((_BR, 1), lambda i: (i, 0)),
            pl.BlockSpec((_BR, 1), lambda i: (i, 0)),
        ],
        out_specs=pl.BlockSpec((1, 1), lambda i: (0, 0)),
        out_shape=jax.ShapeDtypeStruct((1, 1), jnp.float32),
        compiler_params=pltpu.CompilerParams(
            dimension_semantics=("arbitrary",),
        ),
    )(u, v, vself, s0)

    loss_mean = total[0, 0] / n
    return (loss_mean, jnp.asarray(n, dtype=jnp.int32))
